# initial kernel scaffold (unmeasured)
import jax
import jax.numpy as jnp
from jax import lax
from jax.experimental import pallas as pl
from jax.experimental.pallas import tpu as pltpu


def kernel(
    x,
):
    def body(*refs):
        pass

    out_shape = jax.ShapeDtypeStruct(..., jnp.float32)
    return pl.pallas_call(body, out_shape=out_shape)(...)


# baseline (device time: 56883 ns/iter reference)
import functools

import jax
import jax.numpy as jnp
from jax import lax
from jax.experimental import pallas as pl
from jax.experimental.pallas import tpu as pltpu

N_DEV = 8


def kernel(x):
    m, n = x.shape

    def body(x_ref, out_ref, recv_ref, send_sems, recv_sems):
        p = lax.axis_index("i")
        q = lax.rem(p, 4)
        cz = p // 4
        cy = q // 2
        cx = jnp.bitwise_xor(lax.rem(q, 2), cy)
        px = jnp.bitwise_xor(p, 1)
        py = jnp.bitwise_xor(p, 3)
        pz = jnp.bitwise_xor(p, 4)

        barrier_sem = pltpu.get_barrier_semaphore()
        for nbr in (px, py, pz):
            pl.semaphore_signal(
                barrier_sem, inc=1,
                device_id=(nbr,), device_id_type=pl.DeviceIdType.MESH,
            )
        pl.semaphore_wait(barrier_sem, 3)

        out_ref[...] = x_ref[...]

        h2, h4, h8 = m // 2, m // 4, m // 8
        ox = cx * h2
        oy = ox + cy * h4
        oz = oy + cz * h8

        rs_stages = (
            (0, px, (1 - cx) * h2, 0, ox, h2),
            (1, py, ox + (1 - cy) * h4, h2, oy, h4),
            (2, pz, oy + (1 - cz) * h8, h2 + h4, oz, h8),
        )
        for s, partner, send_off, roff, keep_off, size in rs_stages:
            rdma = pltpu.make_async_remote_copy(
                src_ref=out_ref.at[pl.ds(send_off, size), :],
                dst_ref=recv_ref.at[pl.ds(roff, size), :],
                send_sem=send_sems.at[s],
                recv_sem=recv_sems.at[s],
                device_id=(partner,),
                device_id_type=pl.DeviceIdType.MESH,
            )
            rdma.start()
            rdma.wait()
            out_ref[pl.ds(keep_off, size), :] = (
                out_ref[pl.ds(keep_off, size), :]
                + recv_ref[pl.ds(roff, size), :]
            )

        ag_stages = (
            (3, pz, oz, h8),
            (4, py, oy, h4),
            (5, px, ox, h2),
        )
        for s, partner, off, size in ag_stages:
            rdma = pltpu.make_async_remote_copy(
                src_ref=out_ref.at[pl.ds(off, size), :],
                dst_ref=out_ref.at[pl.ds(off, size), :],
                send_sem=send_sems.at[s],
                recv_sem=recv_sems.at[s],
                device_id=(partner,),
                device_id_type=pl.DeviceIdType.MESH,
            )
            rdma.start()
            rdma.wait()

        @functools.partial(
            pl.run_scoped, exit_sem=pltpu.SemaphoreType.REGULAR
        )
        def _(exit_sem):
            for nbr in (px, py, pz):
                pl.semaphore_signal(
                    exit_sem, inc=1,
                    device_id=(nbr,), device_id_type=pl.DeviceIdType.MESH,
                )
            pl.semaphore_wait(exit_sem, 3)

    return pl.pallas_call(
        body,
        out_shape=jax.ShapeDtypeStruct((m, n), x.dtype),
        in_specs=[pl.BlockSpec(memory_space=pltpu.VMEM)],
        out_specs=pl.BlockSpec(memory_space=pltpu.VMEM),
        scratch_shapes=[
            pltpu.VMEM((m // 2 + m // 4 + m // 8, n), x.dtype),
            pltpu.SemaphoreType.DMA((6,)),
            pltpu.SemaphoreType.DMA((6,)),
        ],
        compiler_params=pltpu.CompilerParams(collective_id=0),
    )(x)


# device time: 38841 ns/iter; 1.4645x vs baseline; 1.4645x over previous
import functools

import jax
import jax.numpy as jnp
from jax import lax
from jax.experimental import pallas as pl
from jax.experimental.pallas import tpu as pltpu

N_DEV = 8


def kernel(x):
    m, n = x.shape
    h2, h4, hc = m // 2, m // 4, n // 2

    def body(x_ref, out_ref, recv_ref, send_sems, recv_sems):
        p = lax.axis_index("i")
        q = lax.rem(p, 4)
        cz = p // 4
        cy = q // 2
        cx = jnp.bitwise_xor(lax.rem(q, 2), cy)
        px = jnp.bitwise_xor(p, 1)
        py = jnp.bitwise_xor(p, 3)
        pz = jnp.bitwise_xor(p, 4)

        barrier_sem = pltpu.get_barrier_semaphore()
        for nbr in (px, py, pz):
            pl.semaphore_signal(
                barrier_sem, inc=1,
                device_id=(nbr,), device_id_type=pl.DeviceIdType.MESH,
            )
        pl.semaphore_wait(barrier_sem, 3)

        out_ref[...] = x_ref[...]

        cA = pl.ds(0, hc)
        cB = pl.ds(hc, hc)
        oxA = cx * h2
        oyA = oxA + cy * h4
        ozB = cz * h2
        oyB = ozB + cy * h4

        def rdma(s, partner, src, dst):
            return pltpu.make_async_remote_copy(
                src_ref=src, dst_ref=dst,
                send_sem=send_sems.at[s], recv_sem=recv_sems.at[s],
                device_id=(partner,), device_id_type=pl.DeviceIdType.MESH,
            )

        def add(rows, size, cols, roff):
            out_ref[pl.ds(rows, size), cols] = (
                out_ref[pl.ds(rows, size), cols]
                + recv_ref[pl.ds(roff, size), cols]
            )

        a1 = rdma(0, px, out_ref.at[pl.ds((1 - cx) * h2, h2), cA],
                  recv_ref.at[pl.ds(0, h2), cA])
        b1 = rdma(5, pz, out_ref.at[pl.ds((1 - cz) * h2, h2), cB],
                  recv_ref.at[pl.ds(0, h2), cB])
        a1.start()
        b1.start()

        a1.wait()
        add(oxA, h2, cA, 0)
        a2 = rdma(1, py, out_ref.at[pl.ds(oxA + (1 - cy) * h4, h4), cA],
                  recv_ref.at[pl.ds(h2, h4), cA])
        a2.start()

        b1.wait()
        add(ozB, h2, cB, 0)
        b2 = rdma(6, py, out_ref.at[pl.ds(ozB + (1 - cy) * h4, h4), cB],
                  recv_ref.at[pl.ds(h2, h4), cB])
        b2.start()

        a2.wait()
        add(oyA, h4, cA, h2)
        a3 = rdma(2, pz, out_ref.at[pl.ds(oyA, h4), cA],
                  recv_ref.at[pl.ds(h2 + h4, h4), cA])
        a3.start()

        b2.wait()
        add(oyB, h4, cB, h2)
        b3 = rdma(7, px, out_ref.at[pl.ds(oyB, h4), cB],
                  recv_ref.at[pl.ds(h2 + h4, h4), cB])
        b3.start()

        a3.wait()
        add(oyA, h4, cA, h2 + h4)
        a4 = rdma(3, py, out_ref.at[pl.ds(oyA, h4), cA],
                  out_ref.at[pl.ds(oyA, h4), cA])
        a4.start()

        b3.wait()
        add(oyB, h4, cB, h2 + h4)
        b4 = rdma(8, py, out_ref.at[pl.ds(oyB, h4), cB],
                  out_ref.at[pl.ds(oyB, h4), cB])
        b4.start()

        a4.wait()
        a5 = rdma(4, px, out_ref.at[pl.ds(oxA, h2), cA],
                  out_ref.at[pl.ds(oxA, h2), cA])
        a5.start()

        b4.wait()
        b5 = rdma(9, pz, out_ref.at[pl.ds(ozB, h2), cB],
                  out_ref.at[pl.ds(ozB, h2), cB])
        b5.start()

        a5.wait()
        b5.wait()

        @functools.partial(
            pl.run_scoped, exit_sem=pltpu.SemaphoreType.REGULAR
        )
        def _(exit_sem):
            for nbr in (px, py, pz):
                pl.semaphore_signal(
                    exit_sem, inc=1,
                    device_id=(nbr,), device_id_type=pl.DeviceIdType.MESH,
                )
            pl.semaphore_wait(exit_sem, 3)

    return pl.pallas_call(
        body,
        out_shape=jax.ShapeDtypeStruct((m, n), x.dtype),
        in_specs=[pl.BlockSpec(memory_space=pltpu.VMEM)],
        out_specs=pl.BlockSpec(memory_space=pltpu.VMEM),
        scratch_shapes=[
            pltpu.VMEM((h2 + h4 + h4, n), x.dtype),
            pltpu.SemaphoreType.DMA((10,)),
            pltpu.SemaphoreType.DMA((10,)),
        ],
        compiler_params=pltpu.CompilerParams(collective_id=0),
    )(x)


# device time: 38633 ns/iter; 1.4724x vs baseline; 1.0054x over previous
import functools

import jax
import jax.numpy as jnp
from jax import lax
from jax.experimental import pallas as pl
from jax.experimental.pallas import tpu as pltpu

N_DEV = 8


def kernel(x):
    m, n = x.shape
    h2, h4, hc = m // 2, m // 4, n // 2

    def body(x_ref, out_ref, recv_ref, send_sems, recv_sems):
        p = lax.axis_index("i")
        q = lax.rem(p, 4)
        cz = p // 4
        cy = q // 2
        cx = jnp.bitwise_xor(lax.rem(q, 2), cy)
        px = jnp.bitwise_xor(p, 1)
        py = jnp.bitwise_xor(p, 3)
        pz = jnp.bitwise_xor(p, 4)

        barrier_sem = pltpu.get_barrier_semaphore()
        for nbr in (px, py, pz):
            pl.semaphore_signal(
                barrier_sem, inc=1,
                device_id=(nbr,), device_id_type=pl.DeviceIdType.MESH,
            )
        pl.semaphore_wait(barrier_sem, 3)

        cA = pl.ds(0, hc)
        cB = pl.ds(hc, hc)
        oxA = cx * h2
        oyA = oxA + cy * h4
        ozB = cz * h2
        oyB = ozB + cy * h4

        def rdma(s, partner, src, dst):
            return pltpu.make_async_remote_copy(
                src_ref=src, dst_ref=dst,
                send_sem=send_sems.at[s], recv_sem=recv_sems.at[s],
                device_id=(partner,), device_id_type=pl.DeviceIdType.MESH,
            )

        def add(rows, size, cols, roff):
            out_ref[pl.ds(rows, size), cols] = (
                out_ref[pl.ds(rows, size), cols]
                + recv_ref[pl.ds(roff, size), cols]
            )

        def add_x(rows, size, cols, roff):
            out_ref[pl.ds(rows, size), cols] = (
                x_ref[pl.ds(rows, size), cols]
                + recv_ref[pl.ds(roff, size), cols]
            )

        a1 = rdma(0, px, x_ref.at[pl.ds((1 - cx) * h2, h2), cA],
                  recv_ref.at[pl.ds(0, h2), cA])
        b1 = rdma(5, pz, x_ref.at[pl.ds((1 - cz) * h2, h2), cB],
                  recv_ref.at[pl.ds(0, h2), cB])
        a1.start()
        b1.start()

        sA2 = oxA + (1 - cy) * h4
        sB2 = ozB + (1 - cy) * h4
        a1.wait()
        add_x(sA2, h4, cA, sA2 - oxA)
        a2 = rdma(1, py, out_ref.at[pl.ds(sA2, h4), cA],
                  recv_ref.at[pl.ds(h2, h4), cA])
        a2.start()
        add_x(oyA, h4, cA, oyA - oxA)

        b1.wait()
        add_x(sB2, h4, cB, sB2 - ozB)
        b2 = rdma(6, py, out_ref.at[pl.ds(sB2, h4), cB],
                  recv_ref.at[pl.ds(h2, h4), cB])
        b2.start()
        add_x(oyB, h4, cB, oyB - ozB)

        a2.wait()
        add(oyA, h4, cA, h2)
        a3 = rdma(2, pz, out_ref.at[pl.ds(oyA, h4), cA],
                  recv_ref.at[pl.ds(h2 + h4, h4), cA])
        a3.start()

        b2.wait()
        add(oyB, h4, cB, h2)
        b3 = rdma(7, px, out_ref.at[pl.ds(oyB, h4), cB],
                  recv_ref.at[pl.ds(h2 + h4, h4), cB])
        b3.start()

        a3.wait()
        add(oyA, h4, cA, h2 + h4)
        a4 = rdma(3, py, out_ref.at[pl.ds(oyA, h4), cA],
                  out_ref.at[pl.ds(oyA, h4), cA])
        a4.start()

        b3.wait()
        add(oyB, h4, cB, h2 + h4)
        b4 = rdma(8, py, out_ref.at[pl.ds(oyB, h4), cB],
                  out_ref.at[pl.ds(oyB, h4), cB])
        b4.start()

        a4.wait()
        a5 = rdma(4, px, out_ref.at[pl.ds(oxA, h2), cA],
                  out_ref.at[pl.ds(oxA, h2), cA])
        a5.start()

        b4.wait()
        b5 = rdma(9, pz, out_ref.at[pl.ds(ozB, h2), cB],
                  out_ref.at[pl.ds(ozB, h2), cB])
        b5.start()

        a5.wait()
        b5.wait()

        @functools.partial(
            pl.run_scoped, exit_sem=pltpu.SemaphoreType.REGULAR
        )
        def _(exit_sem):
            for nbr in (px, py, pz):
                pl.semaphore_signal(
                    exit_sem, inc=1,
                    device_id=(nbr,), device_id_type=pl.DeviceIdType.MESH,
                )
            pl.semaphore_wait(exit_sem, 3)

    return pl.pallas_call(
        body,
        out_shape=jax.ShapeDtypeStruct((m, n), x.dtype),
        in_specs=[pl.BlockSpec(memory_space=pltpu.VMEM)],
        out_specs=pl.BlockSpec(memory_space=pltpu.VMEM),
        scratch_shapes=[
            pltpu.VMEM((h2 + h4 + h4, n), x.dtype),
            pltpu.SemaphoreType.DMA((10,)),
            pltpu.SemaphoreType.DMA((10,)),
        ],
        compiler_params=pltpu.CompilerParams(collective_id=0),
    )(x)


# device time: 29749 ns/iter; 1.9121x vs baseline; 1.2986x over previous
import functools

import jax
import jax.numpy as jnp
from jax import lax
from jax.experimental import pallas as pl
from jax.experimental.pallas import tpu as pltpu

N_DEV = 8
BANDS = (0, 352, 704, 1024)


def kernel(x):
    m, n = x.shape

    def body(x_ref, out_ref, recv_ref, send_sems, recv_sems):
        p = lax.axis_index("i")
        q = lax.rem(p, 4)
        cz = p // 4
        cy = q // 2
        cx = jnp.bitwise_xor(lax.rem(q, 2), cy)
        px = jnp.bitwise_xor(p, 1)
        py = jnp.bitwise_xor(p, 3)
        pz = jnp.bitwise_xor(p, 4)

        barrier_sem = pltpu.get_barrier_semaphore()
        for nbr in (px, py, pz):
            pl.semaphore_signal(
                barrier_sem, inc=1,
                device_id=(nbr,), device_id_type=pl.DeviceIdType.MESH,
            )
        pl.semaphore_wait(barrier_sem, 3)

        def rdma(s, partner, src, dst):
            return pltpu.make_async_remote_copy(
                src_ref=src, dst_ref=dst,
                send_sem=send_sems.at[s], recv_sem=recv_sems.at[s],
                device_id=(partner,), device_id_type=pl.DeviceIdType.MESH,
            )

        chains = []
        for i, (c1, p1, c2, p2, p3) in enumerate(
            ((cx, px, cy, py, pz),
             (cy, py, cz, pz, px),
             (cz, pz, cx, px, py))
        ):
            b, r = BANDS[i], BANDS[i + 1] - BANDS[i]
            h, qt = r // 2, r // 4
            chains.append(dict(
                s=5 * i, b=b, h=h, q=qt,
                c1=c1, p1=p1, c2=c2, p2=p2, p3=p3,
                kb1=b + c1 * h,
                kb2=b + c1 * h + c2 * qt,
            ))

        def add(rows, size, roff, first):
            local = x_ref if first else out_ref
            out_ref[pl.ds(rows, size), :] = (
                local[pl.ds(rows, size), :]
                + recv_ref[pl.ds(roff, size), :]
            )

        for c in chains:
            c["r1"] = rdma(
                c["s"], c["p1"],
                x_ref.at[pl.ds(c["b"] + (1 - c["c1"]) * c["h"], c["h"]), :],
                recv_ref.at[pl.ds(c["b"], c["h"]), :],
            )
            c["r1"].start()

        for c in chains:
            c["r1"].wait()
            s2 = c["kb1"] + (1 - c["c2"]) * c["q"]
            add(s2, c["q"], c["b"] + (1 - c["c2"]) * c["q"], first=True)
            c["r2"] = rdma(
                c["s"] + 1, c["p2"],
                out_ref.at[pl.ds(s2, c["q"]), :],
                recv_ref.at[pl.ds(c["b"] + c["h"], c["q"]), :],
            )
            c["r2"].start()
            add(c["kb2"], c["q"], c["b"] + c["c2"] * c["q"], first=True)

        for c in chains:
            c["r2"].wait()
            add(c["kb2"], c["q"], c["b"] + c["h"], first=False)
            c["r3"] = rdma(
                c["s"] + 2, c["p3"],
                out_ref.at[pl.ds(c["kb2"], c["q"]), :],
                recv_ref.at[pl.ds(c["b"] + c["h"] + c["q"], c["q"]), :],
            )
            c["r3"].start()

        for c in chains:
            c["r3"].wait()
            add(c["kb2"], c["q"], c["b"] + c["h"] + c["q"], first=False)
            c["r4"] = rdma(
                c["s"] + 3, c["p2"],
                out_ref.at[pl.ds(c["kb2"], c["q"]), :],
                out_ref.at[pl.ds(c["kb2"], c["q"]), :],
            )
            c["r4"].start()

        for c in chains:
            c["r4"].wait()
            c["r5"] = rdma(
                c["s"] + 4, c["p1"],
                out_ref.at[pl.ds(c["kb1"], c["h"]), :],
                out_ref.at[pl.ds(c["kb1"], c["h"]), :],
            )
            c["r5"].start()

        for c in chains:
            c["r5"].wait()

        @functools.partial(
            pl.run_scoped, exit_sem=pltpu.SemaphoreType.REGULAR
        )
        def _(exit_sem):
            for nbr in (px, py, pz):
                pl.semaphore_signal(
                    exit_sem, inc=1,
                    device_id=(nbr,), device_id_type=pl.DeviceIdType.MESH,
                )
            pl.semaphore_wait(exit_sem, 3)

    return pl.pallas_call(
        body,
        out_shape=jax.ShapeDtypeStruct((m, n), x.dtype),
        in_specs=[pl.BlockSpec(memory_space=pltpu.VMEM)],
        out_specs=pl.BlockSpec(memory_space=pltpu.VMEM),
        scratch_shapes=[
            pltpu.VMEM((m, n), x.dtype),
            pltpu.SemaphoreType.DMA((15,)),
            pltpu.SemaphoreType.DMA((15,)),
        ],
        compiler_params=pltpu.CompilerParams(collective_id=0),
    )(x)


# device time: 25640 ns/iter; 2.2185x vs baseline; 1.1603x over previous
import functools

import jax
import jax.numpy as jnp
from jax import lax
from jax.experimental import pallas as pl
from jax.experimental.pallas import tpu as pltpu

N_DEV = 8
SUB_BANDS = (
    (0, 192), (352, 192), (704, 160),
    (192, 160), (544, 160), (864, 160),
)


def kernel(x):
    m, n = x.shape

    def body(x_ref, out_ref, recv_ref, send_sems, recv_sems):
        p = lax.axis_index("i")
        q = lax.rem(p, 4)
        cz = p // 4
        cy = q // 2
        cx = jnp.bitwise_xor(lax.rem(q, 2), cy)
        px = jnp.bitwise_xor(p, 1)
        py = jnp.bitwise_xor(p, 3)
        pz = jnp.bitwise_xor(p, 4)

        barrier_sem = pltpu.get_barrier_semaphore()
        for nbr in (px, py, pz):
            pl.semaphore_signal(
                barrier_sem, inc=1,
                device_id=(nbr,), device_id_type=pl.DeviceIdType.MESH,
            )
        pl.semaphore_wait(barrier_sem, 3)

        def rdma(s, partner, src, dst):
            return pltpu.make_async_remote_copy(
                src_ref=src, dst_ref=dst,
                send_sem=send_sems.at[s], recv_sem=recv_sems.at[s],
                device_id=(partner,), device_id_type=pl.DeviceIdType.MESH,
            )

        orders = (
            (cx, px, cy, py, pz),
            (cy, py, cz, pz, px),
            (cz, pz, cx, px, py),
        )
        chains = []
        for i, (b, r) in enumerate(SUB_BANDS):
            c1, p1, c2, p2, p3 = orders[i % 3]
            h, qt = r // 2, r // 4
            chains.append(dict(
                s=5 * i, b=b, h=h, q=qt,
                c1=c1, p1=p1, c2=c2, p2=p2, p3=p3,
                kb1=b + c1 * h,
                kb2=b + c1 * h + c2 * qt,
            ))

        def add(rows, size, roff, first):
            local = x_ref if first else out_ref
            out_ref[pl.ds(rows, size), :] = (
                local[pl.ds(rows, size), :]
                + recv_ref[pl.ds(roff, size), :]
            )

        for c in chains:
            c["r1"] = rdma(
                c["s"], c["p1"],
                x_ref.at[pl.ds(c["b"] + (1 - c["c1"]) * c["h"], c["h"]), :],
                recv_ref.at[pl.ds(c["b"], c["h"]), :],
            )
            c["r1"].start()

        for c in chains:
            c["r1"].wait()
            s2 = c["kb1"] + (1 - c["c2"]) * c["q"]
            add(s2, c["q"], c["b"] + (1 - c["c2"]) * c["q"], first=True)
            c["r2"] = rdma(
                c["s"] + 1, c["p2"],
                out_ref.at[pl.ds(s2, c["q"]), :],
                recv_ref.at[pl.ds(c["b"] + c["h"], c["q"]), :],
            )
            c["r2"].start()
            add(c["kb2"], c["q"], c["b"] + c["c2"] * c["q"], first=True)

        for c in chains:
            c["r2"].wait()
            add(c["kb2"], c["q"], c["b"] + c["h"], first=False)
            c["r3"] = rdma(
                c["s"] + 2, c["p3"],
                out_ref.at[pl.ds(c["kb2"], c["q"]), :],
                recv_ref.at[pl.ds(c["b"] + c["h"] + c["q"], c["q"]), :],
            )
            c["r3"].start()

        for c in chains:
            c["r3"].wait()
            add(c["kb2"], c["q"], c["b"] + c["h"] + c["q"], first=False)
            c["r4"] = rdma(
                c["s"] + 3, c["p2"],
                out_ref.at[pl.ds(c["kb2"], c["q"]), :],
                out_ref.at[pl.ds(c["kb2"], c["q"]), :],
            )
            c["r4"].start()

        for c in chains:
            c["r4"].wait()
            c["r5"] = rdma(
                c["s"] + 4, c["p1"],
                out_ref.at[pl.ds(c["kb1"], c["h"]), :],
                out_ref.at[pl.ds(c["kb1"], c["h"]), :],
            )
            c["r5"].start()

        for c in chains:
            c["r5"].wait()

        @functools.partial(
            pl.run_scoped, exit_sem=pltpu.SemaphoreType.REGULAR
        )
        def _(exit_sem):
            for nbr in (px, py, pz):
                pl.semaphore_signal(
                    exit_sem, inc=1,
                    device_id=(nbr,), device_id_type=pl.DeviceIdType.MESH,
                )
            pl.semaphore_wait(exit_sem, 3)

    return pl.pallas_call(
        body,
        out_shape=jax.ShapeDtypeStruct((m, n), x.dtype),
        in_specs=[pl.BlockSpec(memory_space=pltpu.VMEM)],
        out_specs=pl.BlockSpec(memory_space=pltpu.VMEM),
        scratch_shapes=[
            pltpu.VMEM((m, n), x.dtype),
            pltpu.SemaphoreType.DMA((30,)),
            pltpu.SemaphoreType.DMA((30,)),
        ],
        compiler_params=pltpu.CompilerParams(collective_id=0),
    )(x)


# device time: 24480 ns/iter; 2.3237x vs baseline; 1.0474x over previous
import functools

import jax
import jax.numpy as jnp
from jax import lax
from jax.experimental import pallas as pl
from jax.experimental.pallas import tpu as pltpu

N_DEV = 8
SUB_BANDS = (
    (0, 128), (352, 128), (704, 128),
    (128, 128), (480, 128), (832, 96),
    (256, 96), (608, 96), (928, 96),
)


def kernel(x):
    m, n = x.shape

    def body(x_ref, out_ref, recv_ref, send_sems, recv_sems):
        p = lax.axis_index("i")
        q = lax.rem(p, 4)
        cz = p // 4
        cy = q // 2
        cx = jnp.bitwise_xor(lax.rem(q, 2), cy)
        px = jnp.bitwise_xor(p, 1)
        py = jnp.bitwise_xor(p, 3)
        pz = jnp.bitwise_xor(p, 4)

        barrier_sem = pltpu.get_barrier_semaphore()
        for nbr in (px, py, pz):
            pl.semaphore_signal(
                barrier_sem, inc=1,
                device_id=(nbr,), device_id_type=pl.DeviceIdType.MESH,
            )
        pl.semaphore_wait(barrier_sem, 3)

        def rdma(s, partner, src, dst):
            return pltpu.make_async_remote_copy(
                src_ref=src, dst_ref=dst,
                send_sem=send_sems.at[s], recv_sem=recv_sems.at[s],
                device_id=(partner,), device_id_type=pl.DeviceIdType.MESH,
            )

        orders = (
            (cx, px, cy, py, pz),
            (cy, py, cz, pz, px),
            (cz, pz, cx, px, py),
        )
        chains = []
        for i, (b, r) in enumerate(SUB_BANDS):
            c1, p1, c2, p2, p3 = orders[i % 3]
            h, qt = r // 2, r // 4
            chains.append(dict(
                s=5 * i, b=b, h=h, q=qt,
                c1=c1, p1=p1, c2=c2, p2=p2, p3=p3,
                kb1=b + c1 * h,
                kb2=b + c1 * h + c2 * qt,
            ))

        def add(rows, size, roff, first):
            local = x_ref if first else out_ref
            out_ref[pl.ds(rows, size), :] = (
                local[pl.ds(rows, size), :]
                + recv_ref[pl.ds(roff, size), :]
            )

        for c in chains:
            c["r1"] = rdma(
                c["s"], c["p1"],
                x_ref.at[pl.ds(c["b"] + (1 - c["c1"]) * c["h"], c["h"]), :],
                recv_ref.at[pl.ds(c["b"], c["h"]), :],
            )
            c["r1"].start()

        for c in chains:
            c["r1"].wait()
            s2 = c["kb1"] + (1 - c["c2"]) * c["q"]
            add(s2, c["q"], c["b"] + (1 - c["c2"]) * c["q"], first=True)
            c["r2"] = rdma(
                c["s"] + 1, c["p2"],
                out_ref.at[pl.ds(s2, c["q"]), :],
                recv_ref.at[pl.ds(c["b"] + c["h"], c["q"]), :],
            )
            c["r2"].start()
            add(c["kb2"], c["q"], c["b"] + c["c2"] * c["q"], first=True)

        for c in chains:
            c["r2"].wait()
            add(c["kb2"], c["q"], c["b"] + c["h"], first=False)
            c["r3"] = rdma(
                c["s"] + 2, c["p3"],
                out_ref.at[pl.ds(c["kb2"], c["q"]), :],
                recv_ref.at[pl.ds(c["b"] + c["h"] + c["q"], c["q"]), :],
            )
            c["r3"].start()

        for c in chains:
            c["r3"].wait()
            add(c["kb2"], c["q"], c["b"] + c["h"] + c["q"], first=False)
            c["r4"] = rdma(
                c["s"] + 3, c["p2"],
                out_ref.at[pl.ds(c["kb2"], c["q"]), :],
                out_ref.at[pl.ds(c["kb2"], c["q"]), :],
            )
            c["r4"].start()

        for c in chains:
            c["r4"].wait()
            c["r5"] = rdma(
                c["s"] + 4, c["p1"],
                out_ref.at[pl.ds(c["kb1"], c["h"]), :],
                out_ref.at[pl.ds(c["kb1"], c["h"]), :],
            )
            c["r5"].start()

        for c in chains:
            c["r5"].wait()

        @functools.partial(
            pl.run_scoped, exit_sem=pltpu.SemaphoreType.REGULAR
        )
        def _(exit_sem):
            for nbr in (px, py, pz):
                pl.semaphore_signal(
                    exit_sem, inc=1,
                    device_id=(nbr,), device_id_type=pl.DeviceIdType.MESH,
                )
            pl.semaphore_wait(exit_sem, 3)

    return pl.pallas_call(
        body,
        out_shape=jax.ShapeDtypeStruct((m, n), x.dtype),
        in_specs=[pl.BlockSpec(memory_space=pltpu.VMEM)],
        out_specs=pl.BlockSpec(memory_space=pltpu.VMEM),
        scratch_shapes=[
            pltpu.VMEM((m, n), x.dtype),
            pltpu.SemaphoreType.DMA((45,)),
            pltpu.SemaphoreType.DMA((45,)),
        ],
        compiler_params=pltpu.CompilerParams(collective_id=0),
    )(x)


# device time: 24143 ns/iter; 2.3561x vs baseline; 1.0140x over previous
import jax
import jax.numpy as jnp
from jax import lax
from jax.experimental import pallas as pl
from jax.experimental.pallas import tpu as pltpu

N_DEV = 8
SUB_BANDS = (
    (0, 128), (352, 128), (704, 128),
    (128, 128), (480, 128), (832, 96),
    (256, 96), (608, 96), (928, 96),
)


def kernel(x):
    m, n = x.shape

    def body(x_ref, out_ref, recv_ref, send_sems, recv_sems):
        p = lax.axis_index("i")
        q = lax.rem(p, 4)
        cz = p // 4
        cy = q // 2
        cx = jnp.bitwise_xor(lax.rem(q, 2), cy)
        px = jnp.bitwise_xor(p, 1)
        py = jnp.bitwise_xor(p, 3)
        pz = jnp.bitwise_xor(p, 4)

        barrier_sem = pltpu.get_barrier_semaphore()
        for nbr in (px, py, pz):
            pl.semaphore_signal(
                barrier_sem, inc=1,
                device_id=(nbr,), device_id_type=pl.DeviceIdType.MESH,
            )
        pl.semaphore_wait(barrier_sem, 3)

        def rdma(s, partner, src, dst):
            return pltpu.make_async_remote_copy(
                src_ref=src, dst_ref=dst,
                send_sem=send_sems.at[s], recv_sem=recv_sems.at[s],
                device_id=(partner,), device_id_type=pl.DeviceIdType.MESH,
            )

        orders = (
            (cx, px, cy, py, pz),
            (cy, py, cz, pz, px),
            (cz, pz, cx, px, py),
        )
        chains = []
        for i, (b, r) in enumerate(SUB_BANDS):
            c1, p1, c2, p2, p3 = orders[i % 3]
            h, qt = r // 2, r // 4
            chains.append(dict(
                s=5 * i, b=b, h=h, q=qt,
                c1=c1, p1=p1, c2=c2, p2=p2, p3=p3,
                kb1=b + c1 * h,
                kb2=b + c1 * h + c2 * qt,
            ))

        def add(rows, size, roff, first):
            local = x_ref if first else out_ref
            out_ref[pl.ds(rows, size), :] = (
                local[pl.ds(rows, size), :]
                + recv_ref[pl.ds(roff, size), :]
            )

        for c in chains:
            c["r1"] = rdma(
                c["s"], c["p1"],
                x_ref.at[pl.ds(c["b"] + (1 - c["c1"]) * c["h"], c["h"]), :],
                recv_ref.at[pl.ds(c["b"], c["h"]), :],
            )
            c["r1"].start()

        for c in chains:
            c["r1"].wait()
            s2 = c["kb1"] + (1 - c["c2"]) * c["q"]
            add(s2, c["q"], c["b"] + (1 - c["c2"]) * c["q"], first=True)
            c["r2"] = rdma(
                c["s"] + 1, c["p2"],
                out_ref.at[pl.ds(s2, c["q"]), :],
                recv_ref.at[pl.ds(c["b"] + c["h"], c["q"]), :],
            )
            c["r2"].start()
            add(c["kb2"], c["q"], c["b"] + c["c2"] * c["q"], first=True)

        for c in chains:
            c["r2"].wait()
            add(c["kb2"], c["q"], c["b"] + c["h"], first=False)
            c["r3"] = rdma(
                c["s"] + 2, c["p3"],
                out_ref.at[pl.ds(c["kb2"], c["q"]), :],
                recv_ref.at[pl.ds(c["b"] + c["h"] + c["q"], c["q"]), :],
            )
            c["r3"].start()

        for c in chains:
            c["r3"].wait()
            add(c["kb2"], c["q"], c["b"] + c["h"] + c["q"], first=False)
            c["r4"] = rdma(
                c["s"] + 3, c["p2"],
                out_ref.at[pl.ds(c["kb2"], c["q"]), :],
                out_ref.at[pl.ds(c["kb2"], c["q"]), :],
            )
            c["r4"].start()

        for c in chains:
            c["r4"].wait()
            c["r5"] = rdma(
                c["s"] + 4, c["p1"],
                out_ref.at[pl.ds(c["kb1"], c["h"]), :],
                out_ref.at[pl.ds(c["kb1"], c["h"]), :],
            )
            c["r5"].start()

        for c in chains:
            c["r5"].wait()

    return pl.pallas_call(
        body,
        out_shape=jax.ShapeDtypeStruct((m, n), x.dtype),
        in_specs=[pl.BlockSpec(memory_space=pltpu.VMEM)],
        out_specs=pl.BlockSpec(memory_space=pltpu.VMEM),
        scratch_shapes=[
            pltpu.VMEM((m, n), x.dtype),
            pltpu.SemaphoreType.DMA((45,)),
            pltpu.SemaphoreType.DMA((45,)),
        ],
        compiler_params=pltpu.CompilerParams(collective_id=0),
    )(x)


# device time: 23561 ns/iter; 2.4143x vs baseline; 1.0247x over previous
import jax
import jax.numpy as jnp
from jax import lax
from jax.experimental import pallas as pl
from jax.experimental.pallas import tpu as pltpu

N_DEV = 8
SUB_BANDS = (
    (0, 160), (352, 160), (704, 128),
    (160, 128), (512, 128), (832, 128),
    (288, 64), (640, 64), (960, 64),
)


def kernel(x):
    m, n = x.shape

    def body(x_ref, out_ref, recv_ref, send_sems, recv_sems):
        p = lax.axis_index("i")
        q = lax.rem(p, 4)
        cz = p // 4
        cy = q // 2
        cx = jnp.bitwise_xor(lax.rem(q, 2), cy)
        px = jnp.bitwise_xor(p, 1)
        py = jnp.bitwise_xor(p, 3)
        pz = jnp.bitwise_xor(p, 4)

        barrier_sem = pltpu.get_barrier_semaphore()
        for nbr in (px, py, pz):
            pl.semaphore_signal(
                barrier_sem, inc=1,
                device_id=(nbr,), device_id_type=pl.DeviceIdType.MESH,
            )
        pl.semaphore_wait(barrier_sem, 3)

        def rdma(s, partner, src, dst):
            return pltpu.make_async_remote_copy(
                src_ref=src, dst_ref=dst,
                send_sem=send_sems.at[s], recv_sem=recv_sems.at[s],
                device_id=(partner,), device_id_type=pl.DeviceIdType.MESH,
            )

        orders = (
            (cx, px, cy, py, pz),
            (cy, py, cz, pz, px),
            (cz, pz, cx, px, py),
        )
        chains = []
        for i, (b, r) in enumerate(SUB_BANDS):
            c1, p1, c2, p2, p3 = orders[i % 3]
            h, qt = r // 2, r // 4
            chains.append(dict(
                s=5 * i, b=b, h=h, q=qt,
                c1=c1, p1=p1, c2=c2, p2=p2, p3=p3,
                kb1=b + c1 * h,
                kb2=b + c1 * h + c2 * qt,
            ))

        def add(rows, size, roff, first):
            local = x_ref if first else out_ref
            out_ref[pl.ds(rows, size), :] = (
                local[pl.ds(rows, size), :]
                + recv_ref[pl.ds(roff, size), :]
            )

        for c in chains:
            c["r1"] = rdma(
                c["s"], c["p1"],
                x_ref.at[pl.ds(c["b"] + (1 - c["c1"]) * c["h"], c["h"]), :],
                recv_ref.at[pl.ds(c["b"], c["h"]), :],
            )
            c["r1"].start()

        for c in chains:
            c["r1"].wait()
            s2 = c["kb1"] + (1 - c["c2"]) * c["q"]
            add(s2, c["q"], c["b"] + (1 - c["c2"]) * c["q"], first=True)
            c["r2"] = rdma(
                c["s"] + 1, c["p2"],
                out_ref.at[pl.ds(s2, c["q"]), :],
                recv_ref.at[pl.ds(c["b"] + c["h"], c["q"]), :],
            )
            c["r2"].start()
            add(c["kb2"], c["q"], c["b"] + c["c2"] * c["q"], first=True)

        for c in chains:
            c["r2"].wait()
            add(c["kb2"], c["q"], c["b"] + c["h"], first=False)
            c["r3"] = rdma(
                c["s"] + 2, c["p3"],
                out_ref.at[pl.ds(c["kb2"], c["q"]), :],
                recv_ref.at[pl.ds(c["b"] + c["h"] + c["q"], c["q"]), :],
            )
            c["r3"].start()

        for c in chains:
            c["r3"].wait()
            add(c["kb2"], c["q"], c["b"] + c["h"] + c["q"], first=False)
            c["r4"] = rdma(
                c["s"] + 3, c["p2"],
                out_ref.at[pl.ds(c["kb2"], c["q"]), :],
                out_ref.at[pl.ds(c["kb2"], c["q"]), :],
            )
            c["r4"].start()

        for c in chains:
            c["r4"].wait()
            c["r5"] = rdma(
                c["s"] + 4, c["p1"],
                out_ref.at[pl.ds(c["kb1"], c["h"]), :],
                out_ref.at[pl.ds(c["kb1"], c["h"]), :],
            )
            c["r5"].start()

        for c in chains:
            c["r5"].wait()

    return pl.pallas_call(
        body,
        out_shape=jax.ShapeDtypeStruct((m, n), x.dtype),
        in_specs=[pl.BlockSpec(memory_space=pltpu.VMEM)],
        out_specs=pl.BlockSpec(memory_space=pltpu.VMEM),
        scratch_shapes=[
            pltpu.VMEM((m, n), x.dtype),
            pltpu.SemaphoreType.DMA((45,)),
            pltpu.SemaphoreType.DMA((45,)),
        ],
        compiler_params=pltpu.CompilerParams(collective_id=0),
    )(x)
